# baseline (speedup 1.0000x reference)
import functools

import jax
import jax.numpy as jnp
from jax import lax
from jax.experimental import pallas as pl
from jax.experimental.pallas import tpu as pltpu
from jax.experimental.pallas import tpu_sc as plsc

_LANES = 16

_SPLAT_DNUMS = lax.GatherDimensionNumbers(
    offset_dims=(), collapsed_slice_dims=(0,), start_index_map=(0,))


def _splat0(vec):
    idxv = jnp.zeros((_LANES, 1), jnp.int32)
    return lax.gather(vec, idxv, _SPLAT_DNUMS, (1,),
                      mode=lax.GatherScatterMode.PROMISE_IN_BOUNDS)


def _emb_gather_sc(rows, par16, table):
    n_rows = rows.shape[0]
    _, D = table.shape
    info = plsc.get_sparse_core_info()
    nc, ns = info.num_cores, info.num_subcores
    nw = nc * ns
    rows_per_w = n_rows // nw
    pairs_per_w = rows_per_w // 2
    ch = 8
    rch = 2 * ch
    n_chunks = pairs_per_w // ch
    groups = D // _LANES

    mesh = plsc.VectorSubcoreMesh(core_axis_name="c", subcore_axis_name="s")

    @functools.partial(
        pl.kernel,
        mesh=mesh,
        out_type=jax.ShapeDtypeStruct((n_rows, D), table.dtype),
        scratch_types=[
            pltpu.VMEM((rows_per_w,), jnp.int32),
            pltpu.VMEM((rows_per_w + _LANES,), jnp.int32),
            pltpu.VMEM((pairs_per_w * _LANES,), jnp.int32),
            pltpu.VMEM((pairs_per_w * _LANES,), jnp.int32),
            pltpu.VMEM((3, rch, D), jnp.int32),
            pltpu.VMEM((2, ch, D), jnp.int32),
            pltpu.SemaphoreType.DMA,
            pltpu.SemaphoreType.DMA,
            pltpu.SemaphoreType.DMA,
            pltpu.SemaphoreType.DMA,
            pltpu.SemaphoreType.DMA,
        ],
    )
    def k(table_hbm, rows_hbm, par_hbm, out_hbm,
          idx_v, parc_v, sa_v, sb_v, buf, bufO,
          semI, semG0, semG1, semG2, semO):
        tbl32 = table_hbm.bitcast(jnp.int32)
        out32 = out_hbm.bitcast(jnp.int32)
        wid = lax.axis_index("s") * nc + lax.axis_index("c")
        base = wid * rows_per_w
        semG = (semG0, semG1, semG2)

        cpi = pltpu.async_copy(rows_hbm.at[pl.ds(base, rows_per_w)],
                               idx_v, semI)
        cpp = pltpu.async_copy(par_hbm.at[pl.ds(base, rows_per_w)],
                               parc_v.at[pl.ds(0, rows_per_w)], semI)
        cpi.wait()

        def fire_gathers(c):
            s = c % 3
            g1 = pltpu.async_copy(
                tbl32.at[idx_v.at[pl.ds(c * rch, ch)]],
                buf.at[s].at[pl.ds(0, ch)], semG[s])
            g2 = pltpu.async_copy(
                tbl32.at[idx_v.at[pl.ds(c * rch + ch, ch)]],
                buf.at[s].at[pl.ds(ch, ch)], semG[s])
            return g1, g2

        inflight = {0: fire_gathers(0)}
        if n_chunks > 1:
            inflight[1] = fire_gathers(1)
        if n_chunks > 2:
            inflight[2] = fire_gathers(2)
        cpp.wait()

        def replicate(p, _):
            sa_v[pl.ds(p * _LANES, _LANES)] = _splat0(
                parc_v[pl.ds(2 * p, _LANES)])
            sb_v[pl.ds(p * _LANES, _LANES)] = _splat0(
                parc_v[pl.ds(2 * p + 1, _LANES)])
            return 0

        lax.fori_loop(0, pairs_per_w, replicate, 0)

        out_cps = {}
        for c in range(n_chunks):
            s = c % 3
            g1, g2 = inflight.pop(c)
            g1.wait()
            g2.wait()
            if c - 2 in out_cps:
                out_cps.pop(c - 2).wait()

            def blend_pair(p, _, c=c, s=s):
                sa = sa_v[pl.ds((c * ch + p) * _LANES, _LANES)]
                sb = sb_v[pl.ds((c * ch + p) * _LANES, _LANES)]
                for t in range(groups):
                    sl = pl.ds(t * _LANES, _LANES)
                    a = buf[s, 2 * p, sl]
                    b = buf[s, 2 * p + 1, sl]
                    lo = lax.shift_right_logical(a, sa) & 0xFFFF
                    hi = lax.shift_left(lax.shift_right_logical(b, sb), 16)
                    bufO[c % 2, p, sl] = lo | hi
                return 0

            lax.fori_loop(0, ch, blend_pair, 0)
            if c + 3 < n_chunks:
                inflight[c + 3] = fire_gathers(c + 3)
            off2 = pl.multiple_of((base + c * rch) // 2, 8)
            out_cps[c] = pltpu.async_copy(
                bufO.at[c % 2], out32.at[pl.ds(off2, ch)], semO)
        for cp in out_cps.values():
            cp.wait()

    return k(table, rows, par16)


def kernel(xBT, embedding):
    if xBT.ndim == 1:
        xBT = xBT[None, :]
    B, T = xBT.shape
    _, D = embedding.shape
    idx = xBT.reshape(-1).astype(jnp.int32)
    rows = lax.shift_right_logical(idx, 1)
    par16 = lax.shift_left(idx & 1, 4)
    out = _emb_gather_sc(rows, par16, embedding)
    return out.reshape(B, T, D)


# speedup vs baseline: 1.7031x; 1.7031x over previous
import functools

import jax
import jax.numpy as jnp
from jax import lax
from jax.experimental import pallas as pl
from jax.experimental.pallas import tpu as pltpu
from jax.experimental.pallas import tpu_sc as plsc

_LANES = 16

_SPLAT_DNUMS = lax.GatherDimensionNumbers(
    offset_dims=(), collapsed_slice_dims=(0,), start_index_map=(0,))


def _splat0(vec):
    idxv = jnp.zeros((_LANES, 1), jnp.int32)
    return lax.gather(vec, idxv, _SPLAT_DNUMS, (1,),
                      mode=lax.GatherScatterMode.PROMISE_IN_BOUNDS)


def _emb_gather_sc(rowA, parA, rowB, parB, table):
    n_pairs = rowA.shape[0]
    _, D = table.shape
    info = plsc.get_sparse_core_info()
    nc, ns = info.num_cores, info.num_subcores
    nw = nc * ns
    pairs_per_w = n_pairs // nw
    ch = 8
    n_chunks = pairs_per_w // ch
    groups = D // _LANES

    mesh = plsc.VectorSubcoreMesh(core_axis_name="c", subcore_axis_name="s")

    @functools.partial(
        pl.kernel,
        mesh=mesh,
        out_type=jax.ShapeDtypeStruct((2 * n_pairs, D), table.dtype),
        scratch_types=[
            pltpu.VMEM((pairs_per_w,), jnp.int32),
            pltpu.VMEM((pairs_per_w,), jnp.int32),
            pltpu.VMEM((pairs_per_w + _LANES,), jnp.int32),
            pltpu.VMEM((pairs_per_w + _LANES,), jnp.int32),
            pltpu.VMEM((pairs_per_w * _LANES,), jnp.int32),
            pltpu.VMEM((pairs_per_w * _LANES,), jnp.int32),
            pltpu.VMEM((3, ch, D), jnp.int32),
            pltpu.VMEM((3, ch, D), jnp.int32),
            pltpu.VMEM((2, ch, D), jnp.int32),
            pltpu.SemaphoreType.DMA,
            pltpu.SemaphoreType.DMA,
            pltpu.SemaphoreType.DMA,
            pltpu.SemaphoreType.DMA,
            pltpu.SemaphoreType.DMA,
        ],
    )
    def k(table_hbm, rowA_hbm, parA_hbm, rowB_hbm, parB_hbm, out_hbm,
          idxA_v, idxB_v, pA_v, pB_v, sa_v, sb_v, bufA, bufB, bufO,
          semI, semG0, semG1, semG2, semO):
        tbl32 = table_hbm.bitcast(jnp.int32)
        out32 = out_hbm.bitcast(jnp.int32)
        wid = lax.axis_index("s") * nc + lax.axis_index("c")
        base = wid * pairs_per_w
        semG = (semG0, semG1, semG2)

        cps = [
            pltpu.async_copy(rowA_hbm.at[pl.ds(base, pairs_per_w)],
                             idxA_v, semI),
            pltpu.async_copy(rowB_hbm.at[pl.ds(base, pairs_per_w)],
                             idxB_v, semI),
            pltpu.async_copy(parA_hbm.at[pl.ds(base, pairs_per_w)],
                             pA_v.at[pl.ds(0, pairs_per_w)], semI),
            pltpu.async_copy(parB_hbm.at[pl.ds(base, pairs_per_w)],
                             pB_v.at[pl.ds(0, pairs_per_w)], semI),
        ]
        cps[0].wait()
        cps[1].wait()

        def fire_gathers(c):
            s = c % 3
            ga = pltpu.async_copy(
                tbl32.at[idxA_v.at[pl.ds(c * ch, ch)]], bufA.at[s], semG[s])
            gb = pltpu.async_copy(
                tbl32.at[idxB_v.at[pl.ds(c * ch, ch)]], bufB.at[s], semG[s])
            return ga, gb

        inflight = {0: fire_gathers(0)}
        if n_chunks > 1:
            inflight[1] = fire_gathers(1)
        if n_chunks > 2:
            inflight[2] = fire_gathers(2)
        cps[2].wait()
        cps[3].wait()

        def replicate(r, _):
            sa_v[pl.ds(r * _LANES, _LANES)] = _splat0(pA_v[pl.ds(r, _LANES)])
            sb_v[pl.ds(r * _LANES, _LANES)] = _splat0(pB_v[pl.ds(r, _LANES)])
            return 0

        lax.fori_loop(0, pairs_per_w, replicate, 0)

        out_cps = {}
        for c in range(n_chunks):
            s = c % 3
            ga, gb = inflight.pop(c)
            ga.wait()
            gb.wait()
            if c - 2 in out_cps:
                out_cps.pop(c - 2).wait()

            def blend_pair(p, _, c=c, s=s):
                sa = sa_v[pl.ds((c * ch + p) * _LANES, _LANES)]
                sb = sb_v[pl.ds((c * ch + p) * _LANES, _LANES)]
                for t in range(groups):
                    sl = pl.ds(t * _LANES, _LANES)
                    a = bufA[s, p, sl]
                    b = bufB[s, p, sl]
                    lo = lax.shift_right_logical(a, sa) & 0xFFFF
                    hi = lax.shift_left(lax.shift_right_logical(b, sb), 16)
                    bufO[c % 2, p, sl] = lo | hi
                return 0

            lax.fori_loop(0, ch, blend_pair, 0)
            if c + 3 < n_chunks:
                inflight[c + 3] = fire_gathers(c + 3)
            out_cps[c] = pltpu.async_copy(
                bufO.at[c % 2], out32.at[pl.ds(base + c * ch, ch)], semO)
        for cp in out_cps.values():
            cp.wait()

    return k(table, rowA, parA, rowB, parB)


def kernel(xBT, embedding):
    if xBT.ndim == 1:
        xBT = xBT[None, :]
    B, T = xBT.shape
    _, D = embedding.shape
    idx = xBT.reshape(-1).astype(jnp.int32).reshape(-1, 2)
    ia, ib = idx[:, 0], idx[:, 1]
    rowA = lax.shift_right_logical(ia, 1)
    parA = lax.shift_left(ia & 1, 4)
    rowB = lax.shift_right_logical(ib, 1)
    parB = lax.shift_left(ib & 1, 4)
    out = _emb_gather_sc(rowA, parA, rowB, parB, embedding)
    return out.reshape(B, T, D)
